# Initial kernel scaffold; baseline (speedup 1.0000x reference)
import jax
import jax.numpy as jnp
from jax import lax
from jax.experimental import pallas as pl
from jax.experimental.pallas import tpu as pltpu
from jax.experimental.pallas import tpu_sc as plsc

_B = 4
_N = 2048
_K = 16
_L = 16
_NW = 32
_WPB = _NW // _B
_RPW = _N // _WPB
_NCH = _N // _L


def _knn_body(px_hbm, py_hbm, pz_hbm, out_hbm, px, py, pz, obuf):
    c = lax.axis_index("c")
    s = lax.axis_index("s")
    wid = s * 2 + c
    b = wid // _WPB
    base = (wid % _WPB) * _RPW

    pltpu.sync_copy(px_hbm.at[b], px)
    pltpu.sync_copy(py_hbm.at[b], py)
    pltpu.sync_copy(pz_hbm.at[b], pz)

    iota = lax.iota(jnp.int32, _L)
    inf = jnp.full((_L,), jnp.inf, jnp.float32)

    def row_body(r, carry_unused):
        i = base + r
        iv = jnp.broadcast_to(i, (_L,)).astype(jnp.int32)
        qx = plsc.load_gather(px, [iv])
        qy = plsc.load_gather(py, [iv])
        qz = plsc.load_gather(pz, [iv])

        def chunk_body(ch, carry):
            vals, idxs, thr = carry
            off = ch * _L
            dx = px[pl.ds(off, _L)] - qx
            dy = py[pl.ds(off, _L)] - qy
            dz = pz[pl.ds(off, _L)] - qz
            d = dx * dx + dy * dy + dz * dz
            ci = iota + off
            d = jnp.where(ci == iv, jnp.inf, d)

            def do_merge(cr):
                cv, cix, _ = cr
                sd, si = plsc.sort_key_val(d, ci)
                rd = lax.rev(sd, (0,))
                ri = lax.rev(si, (0,))
                take = cv <= rd
                mv = jnp.where(take, cv, rd)
                mi = jnp.where(take, cix, ri)
                nv, ni = plsc.sort_key_val(mv, mi)
                nthr = jnp.broadcast_to(jnp.max(nv), (_L,))
                return nv, ni, nthr

            return lax.cond(jnp.any(d < thr), do_merge, lambda cr: cr,
                            (vals, idxs, thr))

        init = (inf, jnp.zeros((_L,), jnp.int32), inf)
        _, idxs, _ = lax.fori_loop(0, _NCH, chunk_body, init)
        obuf[r, :] = idxs
        return carry_unused

    lax.fori_loop(0, _RPW, row_body, 0)
    pltpu.sync_copy(obuf, out_hbm.at[b, pl.ds(base, _RPW)])


@jax.jit
def kernel(features, points):
    del features
    px = points[..., 0]
    py = points[..., 1]
    pz = points[..., 2]
    mesh = plsc.VectorSubcoreMesh(
        core_axis_name="c", subcore_axis_name="s", num_cores=2, num_subcores=16
    )
    knn = pl.kernel(
        _knn_body,
        out_type=jax.ShapeDtypeStruct((_B, _N, _K), jnp.int32),
        mesh=mesh,
        scratch_types=[
            pltpu.VMEM((_N,), jnp.float32),
            pltpu.VMEM((_N,), jnp.float32),
            pltpu.VMEM((_N,), jnp.float32),
            pltpu.VMEM((_RPW, _K), jnp.int32),
        ],
    )
    topk = knn(px, py, pz)
    bidx = jnp.broadcast_to(
        jnp.arange(_B, dtype=jnp.int32).reshape(_B, 1, 1, 1), (_B, _N, _K, 1)
    )
    return jnp.concatenate([bidx, topk[..., None]], axis=3)


# speedup vs baseline: 2.1867x; 2.1867x over previous
import jax
import jax.numpy as jnp
from jax import lax
from jax.experimental import pallas as pl
from jax.experimental.pallas import tpu as pltpu
from jax.experimental.pallas import tpu_sc as plsc

_B = 4
_N = 2048
_K = 16
_L = 16
_NW = 32
_WPB = _NW // _B
_RPW = _N // _WPB
_NCH = _N // _L


def _knn_body(px_hbm, py_hbm, pz_hbm, out_hbm, px, py, pz, obuf):
    c = lax.axis_index("c")
    s = lax.axis_index("s")
    wid = s * 2 + c
    b = wid // _WPB
    base = (wid % _WPB) * _RPW

    pltpu.sync_copy(px_hbm.at[b], px)
    pltpu.sync_copy(py_hbm.at[b], py)
    pltpu.sync_copy(pz_hbm.at[b], pz)

    iota = lax.iota(jnp.int32, _L)
    inf = jnp.full((_L,), jnp.inf, jnp.float32)

    def row_body(r, carry_unused):
        i = base + r
        iv = jnp.broadcast_to(i, (_L,)).astype(jnp.int32)
        qoff = (i // _L) * _L
        lane = jnp.broadcast_to(i % _L, (_L,)).astype(jnp.int32)
        qx = jnp.take(px[pl.ds(qoff, _L)], lane)
        qy = jnp.take(py[pl.ds(qoff, _L)], lane)
        qz = jnp.take(pz[pl.ds(qoff, _L)], lane)

        def chunk_body(ch, carry):
            vals, idxs, thr = carry
            off = ch * _L
            dx = px[pl.ds(off, _L)] - qx
            dy = py[pl.ds(off, _L)] - qy
            dz = pz[pl.ds(off, _L)] - qz
            d = dx * dx + dy * dy + dz * dz
            ci = iota + off
            d = jnp.where(ci == iv, jnp.inf, d)

            def do_merge(cr):
                cv, cix, _ = cr
                sd, si = plsc.sort_key_val(d, ci)
                rd = lax.rev(sd, (0,))
                ri = lax.rev(si, (0,))
                take = cv <= rd
                mv = jnp.where(take, cv, rd)
                mi = jnp.where(take, cix, ri)
                nv, ni = plsc.sort_key_val(mv, mi)
                nthr = jnp.broadcast_to(jnp.max(nv), (_L,))
                return nv, ni, nthr

            return lax.cond(jnp.any(d < thr), do_merge, lambda cr: cr,
                            (vals, idxs, thr))

        init = (inf, jnp.zeros((_L,), jnp.int32), inf)
        _, idxs, _ = lax.fori_loop(0, _NCH, chunk_body, init)
        obuf[r, :] = idxs
        return carry_unused

    lax.fori_loop(0, _RPW, row_body, 0)
    pltpu.sync_copy(obuf, out_hbm.at[b, pl.ds(base, _RPW)])


@jax.jit
def kernel(features, points):
    del features
    px = points[..., 0]
    py = points[..., 1]
    pz = points[..., 2]
    mesh = plsc.VectorSubcoreMesh(
        core_axis_name="c", subcore_axis_name="s", num_cores=2, num_subcores=16
    )
    knn = pl.kernel(
        _knn_body,
        out_type=jax.ShapeDtypeStruct((_B, _N, _K), jnp.int32),
        mesh=mesh,
        scratch_types=[
            pltpu.VMEM((_N,), jnp.float32),
            pltpu.VMEM((_N,), jnp.float32),
            pltpu.VMEM((_N,), jnp.float32),
            pltpu.VMEM((_RPW, _K), jnp.int32),
        ],
        compiler_params=pltpu.CompilerParams(needs_layout_passes=False),
    )
    topk = knn(px, py, pz)
    bidx = jnp.broadcast_to(
        jnp.arange(_B, dtype=jnp.int32).reshape(_B, 1, 1, 1), (_B, _N, _K, 1)
    )
    return jnp.concatenate([bidx, topk[..., None]], axis=3)
